# baseline (speedup 1.0000x reference)
import functools

import jax
import jax.numpy as jnp
from jax import lax
from jax.experimental import pallas as pl

GROUPS = 4
NUM_Q = 4
K = 1024
DG = 64
TOKENS = 8192
TBLK = 512


def _vq_kernel(x_ref, cb_ref, out_ref, idx_ref, closs_ref):
    i = pl.program_id(0)
    xb = x_ref[...]
    ids = lax.broadcasted_iota(jnp.int32, (TBLK, K), 1)
    ones_row = jnp.ones((1, DG), jnp.float32)

    group_out = []
    closs_cols = []
    for g in range(GROUPS):
        residual = xb[:, g * DG:(g + 1) * DG]
        qout = jnp.zeros_like(residual)
        for q in range(NUM_Q):
            cb = cb_ref[g, q]
            scores = lax.dot_general(
                residual, cb, (((1,), (1,)), ((), ())),
                preferred_element_type=jnp.float32)
            cbsq = cb * cb
            cnorm = lax.dot_general(
                ones_row, cbsq, (((1,), (1,)), ((), ())),
                precision=lax.Precision.HIGHEST,
                preferred_element_type=jnp.float32)
            t = scores - 0.5 * cnorm
            m = jnp.max(t, axis=1, keepdims=True)
            idx = jnp.min(jnp.where(t >= m, ids, K), axis=1, keepdims=True)
            idx_ref[:, pl.ds(g * NUM_Q + q, 1)] = idx
            onehot = (ids == idx).astype(jnp.float32)
            quant = lax.dot_general(
                onehot, cb, (((1,), (0,)), ((), ())),
                precision=lax.Precision.HIGHEST,
                preferred_element_type=jnp.float32)
            new_residual = residual - quant
            closs_cols.append(jnp.sum(new_residual * new_residual))
            qout = qout + quant
            residual = new_residual
        group_out.append(qout)

    out_ref[...] = jnp.concatenate(group_out, axis=1)
    closs_row = jnp.concatenate(
        [jnp.full((1, 1), c, jnp.float32) for c in closs_cols], axis=1)

    @pl.when(i == 0)
    def _():
        closs_ref[...] = closs_row

    @pl.when(i > 0)
    def _():
        closs_ref[...] = closs_ref[...] + closs_row


@jax.jit
def kernel(x, codebooks):
    B, N, D = x.shape
    x2 = x.reshape(TOKENS, D)
    grid = TOKENS // TBLK
    out, idx, closs = pl.pallas_call(
        _vq_kernel,
        grid=(grid,),
        in_specs=[
            pl.BlockSpec((TBLK, D), lambda i: (i, 0)),
            pl.BlockSpec((GROUPS, NUM_Q, K, DG), lambda i: (0, 0, 0, 0)),
        ],
        out_specs=[
            pl.BlockSpec((TBLK, D), lambda i: (i, 0)),
            pl.BlockSpec((TBLK, GROUPS * NUM_Q), lambda i: (i, 0)),
            pl.BlockSpec((1, GROUPS * NUM_Q), lambda i: (0, 0)),
        ],
        out_shape=[
            jax.ShapeDtypeStruct((TOKENS, D), jnp.float32),
            jax.ShapeDtypeStruct((TOKENS, GROUPS * NUM_Q), jnp.int32),
            jax.ShapeDtypeStruct((1, GROUPS * NUM_Q), jnp.float32),
        ],
    )(x2, codebooks)

    quantized = out.reshape(B, N, D)
    all_indices = idx.reshape(B, N, GROUPS, NUM_Q).transpose(2, 0, 1, 3)
    commit_losses = closs.reshape(GROUPS, NUM_Q) / (TOKENS * DG)
    return quantized, all_indices, commit_losses
